# baseline (speedup 1.0000x reference)
import jax
import jax.numpy as jnp
from jax.experimental import pallas as pl

_B = 16
_F = 5
_DS = 64
_C = 62
_T = 128
_G = 32
_H = 4
_DH = 8
_CP = 64
_TT = 8
_CCT = 4
_ES = 512
_ET = 512
_ESP = 576
_ETP = 640
_O = 128
_CH = _C * _G
_CHP = _CP * _G


def _leaky(x):
    return jnp.where(x >= 0, x, 0.2 * x)


def _prep_kernel(srcs_ref, dsts_ref, srct_ref, dstt_ref, wspec_ref, ws_ref,
                 bspec_ref, adsts_ref, adstt_ref,
                 a_s_ref, a_t_ref, wf_ref, bf_ref, mds_ref, mdt_ref):
    dh_s = (jax.lax.broadcasted_iota(jnp.int32, (_C, _ESP), 0)
            == dsts_ref[...]).astype(jnp.float32)
    sh_s = (jax.lax.broadcasted_iota(jnp.int32, (_ESP, _C), 1)
            == srcs_ref[...]).astype(jnp.float32)
    a_s_ref[...] = jnp.dot(dh_s, sh_s, preferred_element_type=jnp.float32)
    dh_t = (jax.lax.broadcasted_iota(jnp.int32, (_T, _ETP), 0)
            == dstt_ref[...]).astype(jnp.float32)
    sh_t = (jax.lax.broadcasted_iota(jnp.int32, (_ETP, _T), 1)
            == srct_ref[...]).astype(jnp.float32)
    a_t_ref[...] = jnp.dot(dh_t, sh_t, preferred_element_type=jnp.float32)
    wf_ref[...] = jnp.dot(wspec_ref[...], ws_ref[...],
                          preferred_element_type=jnp.float32)
    bf_ref[...] = jnp.dot(bspec_ref[...], ws_ref[...],
                          preferred_element_type=jnp.float32)
    rows = jax.lax.broadcasted_iota(jnp.int32, (_G, _H), 0)
    cols = jax.lax.broadcasted_iota(jnp.int32, (_G, _H), 1)
    blk = (rows // _DH == cols).astype(jnp.float32)
    mds_ref[...] = adsts_ref[...] * blk
    mdt_ref[...] = adstt_ref[...] * blk


def _attend(xp, ad, asrc, acnt, mask, bias):
    heads = []
    for h in range(_H):
        xph = xp[:, h * _DH:(h + 1) * _DH]
        asl = jnp.sum(xp[None, :, h * _DH:(h + 1) * _DH]
                      * asrc[:, None, h * _DH:(h + 1) * _DH], axis=-1)
        e = _leaky(ad[:, h:h + 1] + asl)
        m = jnp.max(jnp.where(mask, e, -1e30), axis=1, keepdims=True)
        num = jnp.where(mask, jnp.exp(e - m), 0.0) * acnt
        den = jnp.sum(num, axis=1, keepdims=True)
        alpha = num / (den + 1e-16)
        heads.append(jnp.dot(alpha, xph, preferred_element_type=jnp.float32))
    return _leaky(jnp.concatenate(heads, axis=1) + bias)


def _spatial_kernel(x_ref, acnt_ref, wf_ref, bf_ref, mds_ref, asrc_ref,
                    bias_ref, out_ref):
    xb = x_ref[0]
    xp = jnp.dot(xb.reshape(_TT * _C, _F), wf_ref[...],
                 preferred_element_type=jnp.float32) + bf_ref[...]
    ad = jnp.dot(xp, mds_ref[...], preferred_element_type=jnp.float32)
    xp3 = xp.reshape(_TT, _C, _G)
    ad3 = ad.reshape(_TT, _C, _H)
    acnt = acnt_ref[...]
    mask = acnt > 0.0
    asrc = asrc_ref[...]
    bias = bias_ref[...]
    outs = []
    for t in range(_TT):
        g = _attend(xp3[t], ad3[t], asrc, acnt, mask, bias)
        g = jnp.concatenate(
            [g, jnp.zeros((_CP - _C, _G), jnp.float32)], axis=0)
        outs.append(g[:, None, :])
    out_ref[0] = jnp.concatenate(outs, axis=1)


def _temporal_kernel(gs_ref, acnt_ref, wt_ref, mdt_ref, asrc_ref, bias_ref,
                     out_ref):
    acnt = acnt_ref[...]
    mask = acnt > 0.0
    asrc = asrc_ref[...]
    bias = bias_ref[...]
    cols = []
    for c in range(_CCT):
        xin = gs_ref[0, c]
        xp = jnp.dot(xin, wt_ref[...], preferred_element_type=jnp.float32)
        ad = jnp.dot(xp, mdt_ref[...], preferred_element_type=jnp.float32)
        cols.append(_attend(xp, ad, asrc, acnt, mask, bias))
    out_ref[0] = jnp.concatenate(cols, axis=1)


def _shift_rows(x, s):
    if s == 0:
        return x
    return jnp.concatenate(
        [jnp.zeros((s, x.shape[1]), x.dtype), x[:-s]], axis=0)


def _causal_conv(xin, w_ref, b, d):
    acc = jnp.dot(_shift_rows(xin, 2 * d), w_ref[0],
                  preferred_element_type=jnp.float32)
    acc = acc + jnp.dot(_shift_rows(xin, d), w_ref[1],
                        preferred_element_type=jnp.float32)
    acc = acc + jnp.dot(xin, w_ref[2], preferred_element_type=jnp.float32)
    return acc + b


def _tcn_kernel(x_ref, w1a_ref, w1b_ref, dw_ref, w2a_ref, w2b_ref,
                b1a_ref, b1b_ref, db_ref, g1_ref, be1_ref, m1_ref, v1_ref,
                b2a_ref, b2b_ref, g2_ref, be2_ref, m2_ref, v2_ref, out_ref):
    x = x_ref[0]
    res = jnp.dot(x, dw_ref[...], preferred_element_type=jnp.float32) \
        + db_ref[...]
    h = jax.nn.relu(_causal_conv(x, w1a_ref, b1a_ref[...], 1))
    h = jax.nn.relu(_causal_conv(h, w1b_ref, b1b_ref[...], 1))
    h = h + res
    scale1 = g1_ref[...] * jax.lax.rsqrt(v1_ref[...] + 1e-5)
    h = (h - m1_ref[...]) * scale1 + be1_ref[...]
    res2 = h
    h = jax.nn.relu(_causal_conv(h, w2a_ref, b2a_ref[...], 2))
    h = jax.nn.relu(_causal_conv(h, w2b_ref, b2b_ref[...], 2))
    h = h + res2
    scale2 = g2_ref[...] * jax.lax.rsqrt(v2_ref[...] + 1e-5)
    h = (h - m2_ref[...]) * scale2 + be2_ref[...]
    out_ref[0] = jnp.max(h, axis=0, keepdims=True)


def kernel(x, spatial_edge_index, temporal_edge_index, W_spec, b_spec, Ws,
           asrc_s, adst_s, bias_s, Wt, asrc_t, adst_t, bias_t,
           tb1_w1, tb1_b1, tb1_w2, tb1_b2, tb1_dw, tb1_db,
           tb1_gamma, tb1_beta, tb1_mean, tb1_var,
           tb2_w1, tb2_b1, tb2_w2, tb2_b2,
           tb2_gamma, tb2_beta, tb2_mean, tb2_var):
    f32 = jnp.float32
    idt = spatial_edge_index.dtype

    sl_c = jnp.arange(_C, dtype=idt)
    sl_t = jnp.arange(_T, dtype=idt)
    pad_s = jnp.full((_ESP - _ES - _C,), -1, idt)
    src_s = jnp.concatenate([spatial_edge_index[0], sl_c, pad_s])
    dst_s = jnp.concatenate([spatial_edge_index[1], sl_c, pad_s])
    pad_t = jnp.full((_ETP - _ET - _T,), -1, idt)
    src_t = jnp.concatenate([temporal_edge_index[0], sl_t, pad_t])
    dst_t = jnp.concatenate([temporal_edge_index[1], sl_t, pad_t])

    a_s, a_t, wf, bf, mds, mdt = pl.pallas_call(
        _prep_kernel,
        out_shape=(
            jax.ShapeDtypeStruct((_C, _C), f32),
            jax.ShapeDtypeStruct((_T, _T), f32),
            jax.ShapeDtypeStruct((_F, _G), f32),
            jax.ShapeDtypeStruct((1, _G), f32),
            jax.ShapeDtypeStruct((_G, _H), f32),
            jax.ShapeDtypeStruct((_G, _H), f32),
        ),
    )(src_s.reshape(_ESP, 1), dst_s.reshape(1, _ESP),
      src_t.reshape(_ETP, 1), dst_t.reshape(1, _ETP),
      W_spec, Ws, b_spec.reshape(1, _DS),
      adst_s.reshape(_G, 1), adst_t.reshape(_G, 1))

    xT = jnp.transpose(x, (0, 3, 2, 1))
    gs = pl.pallas_call(
        _spatial_kernel,
        grid=(_B, _T // _TT),
        in_specs=[
            pl.BlockSpec((1, _TT, _C, _F), lambda b, t: (b, t, 0, 0)),
            pl.BlockSpec((_C, _C), lambda b, t: (0, 0)),
            pl.BlockSpec((_F, _G), lambda b, t: (0, 0)),
            pl.BlockSpec((1, _G), lambda b, t: (0, 0)),
            pl.BlockSpec((_G, _H), lambda b, t: (0, 0)),
            pl.BlockSpec((1, _G), lambda b, t: (0, 0)),
            pl.BlockSpec((1, _G), lambda b, t: (0, 0)),
        ],
        out_specs=pl.BlockSpec((1, _CP, _TT, _G), lambda b, t: (b, 0, t, 0)),
        out_shape=jax.ShapeDtypeStruct((_B, _CP, _T, _G), f32),
    )(xT, a_s, wf, bf, mds, asrc_s.reshape(1, _G), bias_s.reshape(1, _G))

    tcnin = pl.pallas_call(
        _temporal_kernel,
        grid=(_B, _CP // _CCT),
        in_specs=[
            pl.BlockSpec((1, _CCT, _T, _G), lambda b, c: (b, c, 0, 0)),
            pl.BlockSpec((_T, _T), lambda b, c: (0, 0)),
            pl.BlockSpec((_G, _G), lambda b, c: (0, 0)),
            pl.BlockSpec((_G, _H), lambda b, c: (0, 0)),
            pl.BlockSpec((1, _G), lambda b, c: (0, 0)),
            pl.BlockSpec((1, _G), lambda b, c: (0, 0)),
        ],
        out_specs=pl.BlockSpec((1, _T, _CCT * _G), lambda b, c: (b, 0, c)),
        out_shape=jax.ShapeDtypeStruct((_B, _T, _CHP), f32),
    )(gs, a_t, Wt, mdt, asrc_t.reshape(1, _G), bias_t.reshape(1, _G))

    zpad = jnp.zeros((3, _CHP - _CH, _O), f32)
    w1a = jnp.concatenate([jnp.transpose(tb1_w1, (2, 1, 0)), zpad], axis=1)
    w1b = jnp.transpose(tb1_w2, (2, 1, 0))
    dw = jnp.concatenate(
        [jnp.transpose(tb1_dw[:, :, 0], (1, 0)),
         jnp.zeros((_CHP - _CH, _O), f32)], axis=0)
    w2a = jnp.transpose(tb2_w1, (2, 1, 0))
    w2b = jnp.transpose(tb2_w2, (2, 1, 0))

    def row(v):
        return v.reshape(1, _O)

    def full(*shape):
        return [pl.BlockSpec(shape, lambda b: tuple(0 for _ in shape))]

    out = pl.pallas_call(
        _tcn_kernel,
        grid=(_B,),
        in_specs=(
            [pl.BlockSpec((1, _T, _CHP), lambda b: (b, 0, 0))]
            + full(3, _CHP, _O) + full(3, _O, _O) + full(_CHP, _O)
            + full(3, _O, _O) + full(3, _O, _O)
            + full(1, _O) * 13
        ),
        out_specs=pl.BlockSpec((1, 1, _O), lambda b: (b, 0, 0)),
        out_shape=jax.ShapeDtypeStruct((_B, 1, _O), f32),
    )(tcnin, w1a, w1b, dw, w2a, w2b,
      row(tb1_b1), row(tb1_b2), row(tb1_db),
      row(tb1_gamma), row(tb1_beta), row(tb1_mean), row(tb1_var),
      row(tb2_b1), row(tb2_b2),
      row(tb2_gamma), row(tb2_beta), row(tb2_mean), row(tb2_var))

    return out.reshape(_B, _O)


# speedup vs baseline: 3.5336x; 3.5336x over previous
import jax
import jax.numpy as jnp
from jax.experimental import pallas as pl

_B = 16
_F = 5
_DS = 64
_C = 62
_T = 128
_G = 32
_H = 4
_DH = 8
_CP = 64
_TT = 8
_CCT = 4
_ES = 512
_ET = 512
_ESP = 576
_ETP = 640
_O = 128
_CH = _C * _G
_CHP = _CP * _G


def _leaky(x):
    return jnp.where(x >= 0, x, 0.2 * x)


def _prep_kernel(srcs_ref, dsts_ref, srct_ref, dstt_ref, wspec_ref, ws_ref,
                 bspec_ref, adsts_ref, adstt_ref,
                 a_s_ref, a_t_ref, wf_ref, bf_ref, mds_ref, mdt_ref):
    dh_s = (jax.lax.broadcasted_iota(jnp.int32, (_C, _ESP), 0)
            == dsts_ref[...]).astype(jnp.float32)
    sh_s = (jax.lax.broadcasted_iota(jnp.int32, (_ESP, _C), 1)
            == srcs_ref[...]).astype(jnp.float32)
    a_s_ref[...] = jnp.dot(dh_s, sh_s, preferred_element_type=jnp.float32)
    dh_t = (jax.lax.broadcasted_iota(jnp.int32, (_T, _ETP), 0)
            == dstt_ref[...]).astype(jnp.float32)
    sh_t = (jax.lax.broadcasted_iota(jnp.int32, (_ETP, _T), 1)
            == srct_ref[...]).astype(jnp.float32)
    a_t_ref[...] = jnp.dot(dh_t, sh_t, preferred_element_type=jnp.float32)
    wf_ref[...] = jnp.dot(wspec_ref[...], ws_ref[...],
                          preferred_element_type=jnp.float32)
    bf_ref[...] = jnp.dot(bspec_ref[...], ws_ref[...],
                          preferred_element_type=jnp.float32)
    rows = jax.lax.broadcasted_iota(jnp.int32, (_G, _H), 0)
    cols = jax.lax.broadcasted_iota(jnp.int32, (_G, _H), 1)
    blk = (rows // _DH == cols).astype(jnp.float32)
    mds_ref[...] = adsts_ref[...] * blk
    mdt_ref[...] = adstt_ref[...] * blk


def _gat_tile(xp3, ad3, asrc, acnt, bias, nb, n):
    asl = [jnp.sum(xp3[:, :, h * _DH:(h + 1) * _DH]
                   * asrc[:, None, h * _DH:(h + 1) * _DH], axis=-1)
           for h in range(_H)]
    acnt2 = jnp.concatenate([acnt, acnt], axis=1)
    num_pairs = []
    den = []
    for p in range(_H // 2):
        h0, h1 = 2 * p, 2 * p + 1
        aslp = jnp.concatenate([asl[h0], asl[h1]], axis=1)[:, None, :]
        e1s = jnp.exp(aslp)
        e2s = jnp.exp(0.2 * aslp)
        adh0 = ad3[:, :, h0:h0 + 1]
        adh1 = ad3[:, :, h1:h1 + 1]
        adp = jnp.concatenate([jnp.broadcast_to(adh0, (nb, n, n)),
                               jnp.broadcast_to(adh1, (nb, n, n))], axis=2)
        e1d = jnp.concatenate(
            [jnp.broadcast_to(jnp.exp(adh0), (nb, n, n)),
             jnp.broadcast_to(jnp.exp(adh1), (nb, n, n))], axis=2)
        e2d = jnp.concatenate(
            [jnp.broadcast_to(jnp.exp(0.2 * adh0), (nb, n, n)),
             jnp.broadcast_to(jnp.exp(0.2 * adh1), (nb, n, n))], axis=2)
        pos = (adp + aslp) >= 0
        num = acnt2 * jnp.where(pos, e1d, e2d) * jnp.where(pos, e1s, e2s)
        num_pairs.append(num)
        den.append(jnp.sum(num[:, :, :n], axis=2, keepdims=True))
        den.append(jnp.sum(num[:, :, n:], axis=2, keepdims=True))
    rec = [1.0 / (d + 1e-16) for d in den]
    hmask = [(jax.lax.broadcasted_iota(jnp.int32, (1, _G), 1) // _DH == h
              ).astype(jnp.float32) for h in range(_H)]
    outs = []
    for b in range(nb):
        pmat = jnp.concatenate([num_pairs[0][b], num_pairs[1][b]], axis=1)
        xv = jnp.concatenate([xp3[b] * hmask[h] for h in range(_H)], axis=0)
        o = jnp.dot(pmat, xv, preferred_element_type=jnp.float32)
        r = jnp.concatenate(
            [jnp.broadcast_to(rec[h][b], (n, _DH)) for h in range(_H)], axis=1)
        outs.append(_leaky(o * r + bias))
    return outs


def _spatial_kernel(x_ref, acnt_ref, wf_ref, bf_ref, mds_ref, asrc_ref,
                    bias_ref, out_ref):
    xb = x_ref[0]
    xp = jnp.dot(xb.reshape(_TT * _C, _F), wf_ref[...],
                 preferred_element_type=jnp.float32) + bf_ref[...]
    ad = jnp.dot(xp, mds_ref[...], preferred_element_type=jnp.float32)
    xp3 = xp.reshape(_TT, _C, _G)
    ad3 = ad.reshape(_TT, _C, _H)
    outs = _gat_tile(xp3, ad3, asrc_ref[...], acnt_ref[...], bias_ref[...],
                     _TT, _C)
    zp = jnp.zeros((_CP - _C, _G), jnp.float32)
    out_ref[0] = jnp.concatenate(
        [jnp.concatenate([g, zp], axis=0)[:, None, :] for g in outs], axis=1)


def _temporal_kernel(gs_ref, acnt_ref, wt_ref, mdt_ref, asrc_ref, bias_ref,
                     out_ref):
    xin = gs_ref[0]
    xp = jnp.dot(xin.reshape(_CCT * _T, _G), wt_ref[...],
                 preferred_element_type=jnp.float32)
    ad = jnp.dot(xp, mdt_ref[...], preferred_element_type=jnp.float32)
    xp3 = xp.reshape(_CCT, _T, _G)
    ad3 = ad.reshape(_CCT, _T, _H)
    outs = _gat_tile(xp3, ad3, asrc_ref[...], acnt_ref[...], bias_ref[...],
                     _CCT, _T)
    out_ref[0] = jnp.concatenate(outs, axis=1)


def _shift_rows(x, s):
    if s == 0:
        return x
    return jnp.concatenate(
        [jnp.zeros((s, x.shape[1]), x.dtype), x[:-s]], axis=0)


def _causal_conv(xin, w_ref, b, d):
    acc = jnp.dot(_shift_rows(xin, 2 * d), w_ref[0],
                  preferred_element_type=jnp.float32)
    acc = acc + jnp.dot(_shift_rows(xin, d), w_ref[1],
                        preferred_element_type=jnp.float32)
    acc = acc + jnp.dot(xin, w_ref[2], preferred_element_type=jnp.float32)
    return acc + b


def _tcn_kernel(x_ref, w1a_ref, w1b_ref, dw_ref, w2a_ref, w2b_ref,
                b1a_ref, b1b_ref, db_ref, g1_ref, be1_ref, m1_ref, v1_ref,
                b2a_ref, b2b_ref, g2_ref, be2_ref, m2_ref, v2_ref, out_ref):
    x = x_ref[0]
    res = jnp.dot(x, dw_ref[...], preferred_element_type=jnp.float32) \
        + db_ref[...]
    h = jax.nn.relu(_causal_conv(x, w1a_ref, b1a_ref[...], 1))
    h = jax.nn.relu(_causal_conv(h, w1b_ref, b1b_ref[...], 1))
    h = h + res
    scale1 = g1_ref[...] * jax.lax.rsqrt(v1_ref[...] + 1e-5)
    h = (h - m1_ref[...]) * scale1 + be1_ref[...]
    res2 = h
    h = jax.nn.relu(_causal_conv(h, w2a_ref, b2a_ref[...], 2))
    h = jax.nn.relu(_causal_conv(h, w2b_ref, b2b_ref[...], 2))
    h = h + res2
    scale2 = g2_ref[...] * jax.lax.rsqrt(v2_ref[...] + 1e-5)
    h = (h - m2_ref[...]) * scale2 + be2_ref[...]
    out_ref[0] = jnp.max(h, axis=0, keepdims=True)


def kernel(x, spatial_edge_index, temporal_edge_index, W_spec, b_spec, Ws,
           asrc_s, adst_s, bias_s, Wt, asrc_t, adst_t, bias_t,
           tb1_w1, tb1_b1, tb1_w2, tb1_b2, tb1_dw, tb1_db,
           tb1_gamma, tb1_beta, tb1_mean, tb1_var,
           tb2_w1, tb2_b1, tb2_w2, tb2_b2,
           tb2_gamma, tb2_beta, tb2_mean, tb2_var):
    f32 = jnp.float32
    idt = spatial_edge_index.dtype

    sl_c = jnp.arange(_C, dtype=idt)
    sl_t = jnp.arange(_T, dtype=idt)
    pad_s = jnp.full((_ESP - _ES - _C,), -1, idt)
    src_s = jnp.concatenate([spatial_edge_index[0], sl_c, pad_s])
    dst_s = jnp.concatenate([spatial_edge_index[1], sl_c, pad_s])
    pad_t = jnp.full((_ETP - _ET - _T,), -1, idt)
    src_t = jnp.concatenate([temporal_edge_index[0], sl_t, pad_t])
    dst_t = jnp.concatenate([temporal_edge_index[1], sl_t, pad_t])

    a_s, a_t, wf, bf, mds, mdt = pl.pallas_call(
        _prep_kernel,
        out_shape=(
            jax.ShapeDtypeStruct((_C, _C), f32),
            jax.ShapeDtypeStruct((_T, _T), f32),
            jax.ShapeDtypeStruct((_F, _G), f32),
            jax.ShapeDtypeStruct((1, _G), f32),
            jax.ShapeDtypeStruct((_G, _H), f32),
            jax.ShapeDtypeStruct((_G, _H), f32),
        ),
    )(src_s.reshape(_ESP, 1), dst_s.reshape(1, _ESP),
      src_t.reshape(_ETP, 1), dst_t.reshape(1, _ETP),
      W_spec, Ws, b_spec.reshape(1, _DS),
      adst_s.reshape(_G, 1), adst_t.reshape(_G, 1))

    xT = jnp.transpose(x, (0, 3, 2, 1))
    gs = pl.pallas_call(
        _spatial_kernel,
        grid=(_B, _T // _TT),
        in_specs=[
            pl.BlockSpec((1, _TT, _C, _F), lambda b, t: (b, t, 0, 0)),
            pl.BlockSpec((_C, _C), lambda b, t: (0, 0)),
            pl.BlockSpec((_F, _G), lambda b, t: (0, 0)),
            pl.BlockSpec((1, _G), lambda b, t: (0, 0)),
            pl.BlockSpec((_G, _H), lambda b, t: (0, 0)),
            pl.BlockSpec((1, _G), lambda b, t: (0, 0)),
            pl.BlockSpec((1, _G), lambda b, t: (0, 0)),
        ],
        out_specs=pl.BlockSpec((1, _CP, _TT, _G), lambda b, t: (b, 0, t, 0)),
        out_shape=jax.ShapeDtypeStruct((_B, _CP, _T, _G), f32),
    )(xT, a_s, wf, bf, mds, asrc_s.reshape(1, _G), bias_s.reshape(1, _G))

    tcnin = pl.pallas_call(
        _temporal_kernel,
        grid=(_B, _CP // _CCT),
        in_specs=[
            pl.BlockSpec((1, _CCT, _T, _G), lambda b, c: (b, c, 0, 0)),
            pl.BlockSpec((_T, _T), lambda b, c: (0, 0)),
            pl.BlockSpec((_G, _G), lambda b, c: (0, 0)),
            pl.BlockSpec((_G, _H), lambda b, c: (0, 0)),
            pl.BlockSpec((1, _G), lambda b, c: (0, 0)),
            pl.BlockSpec((1, _G), lambda b, c: (0, 0)),
        ],
        out_specs=pl.BlockSpec((1, _T, _CCT * _G), lambda b, c: (b, 0, c)),
        out_shape=jax.ShapeDtypeStruct((_B, _T, _CHP), f32),
    )(gs, a_t, Wt, mdt, asrc_t.reshape(1, _G), bias_t.reshape(1, _G))

    zpad = jnp.zeros((3, _CHP - _CH, _O), f32)
    w1a = jnp.concatenate([jnp.transpose(tb1_w1, (2, 1, 0)), zpad], axis=1)
    w1b = jnp.transpose(tb1_w2, (2, 1, 0))
    dw = jnp.concatenate(
        [jnp.transpose(tb1_dw[:, :, 0], (1, 0)),
         jnp.zeros((_CHP - _CH, _O), f32)], axis=0)
    w2a = jnp.transpose(tb2_w1, (2, 1, 0))
    w2b = jnp.transpose(tb2_w2, (2, 1, 0))

    def row(v):
        return v.reshape(1, _O)

    def full(*shape):
        return [pl.BlockSpec(shape, lambda b: tuple(0 for _ in shape))]

    out = pl.pallas_call(
        _tcn_kernel,
        grid=(_B,),
        in_specs=(
            [pl.BlockSpec((1, _T, _CHP), lambda b: (b, 0, 0))]
            + full(3, _CHP, _O) + full(3, _O, _O) + full(_CHP, _O)
            + full(3, _O, _O) + full(3, _O, _O)
            + full(1, _O) * 13
        ),
        out_specs=pl.BlockSpec((1, 1, _O), lambda b: (b, 0, 0)),
        out_shape=jax.ShapeDtypeStruct((_B, 1, _O), f32),
    )(tcnin, w1a, w1b, dw, w2a, w2b,
      row(tb1_b1), row(tb1_b2), row(tb1_db),
      row(tb1_gamma), row(tb1_beta), row(tb1_mean), row(tb1_var),
      row(tb2_b1), row(tb2_b2),
      row(tb2_gamma), row(tb2_beta), row(tb2_mean), row(tb2_var))

    return out.reshape(_B, _O)


# speedup vs baseline: 4.4239x; 1.2519x over previous
import jax
import jax.numpy as jnp
from jax.experimental import pallas as pl

_B = 16
_F = 5
_DS = 64
_C = 62
_T = 128
_G = 32
_H = 4
_DH = 8
_CP = 64
_TT = 8
_CCT = 4
_ES = 512
_ET = 512
_ESP = 576
_ETP = 640
_O = 128
_CH = _C * _G
_CHP = _CP * _G


def _leaky(x):
    return jnp.where(x >= 0, x, 0.2 * x)


def _prep_kernel(srcs_ref, dsts_ref, srct_ref, dstt_ref, wspec_ref, ws_ref,
                 bspec_ref, asrcs_ref, asrct_ref, adsts_ref, adstt_ref,
                 a_s_ref, a_t_ref, wf_ref, bf_ref, mss_ref, mst_ref,
                 mds_ref, mdt_ref):
    dh_s = (jax.lax.broadcasted_iota(jnp.int32, (_C, _ESP), 0)
            == dsts_ref[...]).astype(jnp.float32)
    sh_s = (jax.lax.broadcasted_iota(jnp.int32, (_ESP, _T), 1)
            == srcs_ref[...]).astype(jnp.float32)
    a_s_ref[...] = jnp.dot(dh_s, sh_s, preferred_element_type=jnp.float32)
    dh_t = (jax.lax.broadcasted_iota(jnp.int32, (_T, _ETP), 0)
            == dstt_ref[...]).astype(jnp.float32)
    sh_t = (jax.lax.broadcasted_iota(jnp.int32, (_ETP, _T), 1)
            == srct_ref[...]).astype(jnp.float32)
    a_t_ref[...] = jnp.dot(dh_t, sh_t, preferred_element_type=jnp.float32)
    wf_ref[...] = jnp.dot(wspec_ref[...], ws_ref[...],
                          preferred_element_type=jnp.float32)
    bf_ref[...] = jnp.dot(bspec_ref[...], ws_ref[...],
                          preferred_element_type=jnp.float32)
    rows = jax.lax.broadcasted_iota(jnp.int32, (_G, _H), 0)
    cols = jax.lax.broadcasted_iota(jnp.int32, (_G, _H), 1)
    blk = (rows // _DH == cols).astype(jnp.float32)
    mss_ref[...] = asrcs_ref[...] * blk
    mst_ref[...] = asrct_ref[...] * blk
    mds_ref[...] = adsts_ref[...] * blk
    mdt_ref[...] = adstt_ref[...] * blk


def _hmasks():
    return [(jax.lax.broadcasted_iota(jnp.int32, (1, _G), 1) // _DH == h
             ).astype(jnp.float32) for h in range(_H)]


def _gat_tile(xp3, asm3, ad3, acnt, bias, nb, n, npad):
    asT = jnp.transpose(asm3, (0, 2, 1))
    if npad != n:
        asT = jnp.concatenate(
            [asT, jnp.zeros((nb, _H, npad - n), jnp.float32)], axis=2)
    nums, dens = [], []
    for h in range(_H):
        arow = asT[:, h:h + 1, :]
        adh = ad3[:, :, h:h + 1]
        pos = (adh + arow) >= 0
        seld = jnp.where(pos, jnp.exp(adh), jnp.exp(0.2 * adh))
        sels = jnp.where(pos, jnp.exp(arow), jnp.exp(0.2 * arow))
        num = acnt * seld * sels
        nums.append(num)
        dens.append(jnp.sum(num, axis=2, keepdims=True))
    numcat = jnp.concatenate(nums, axis=2)
    recc = 1.0 / (jnp.concatenate(dens, axis=2) + 1e-16)
    hmask = _hmasks()
    r8 = jnp.concatenate(hmask, axis=0)
    zrow = None if npad == n else jnp.zeros((npad - n, _G), jnp.float32)
    outs = []
    for b in range(nb):
        if zrow is None:
            xv = jnp.concatenate(
                [xp3[b] * hmask[h] for h in range(_H)], axis=0)
        else:
            xv = jnp.concatenate(
                [jnp.concatenate([xp3[b] * hmask[h], zrow], axis=0)
                 for h in range(_H)], axis=0)
        o = jnp.dot(numcat[b], xv, preferred_element_type=jnp.float32)
        r = jnp.dot(recc[b], r8, preferred_element_type=jnp.float32)
        outs.append(_leaky(o * r + bias))
    return outs


def _spatial_kernel(x_ref, acnt_ref, wf_ref, bf_ref, mss_ref, mds_ref,
                    bias_ref, out_ref):
    xb = x_ref[0]
    xp = jnp.dot(xb.reshape(_TT * _C, _F), wf_ref[...],
                 preferred_element_type=jnp.float32) + bf_ref[...]
    asm = jnp.dot(xp, mss_ref[...], preferred_element_type=jnp.float32)
    ad = jnp.dot(xp, mds_ref[...], preferred_element_type=jnp.float32)
    xp3 = xp.reshape(_TT, _C, _G)
    outs = _gat_tile(xp3, asm.reshape(_TT, _C, _H), ad.reshape(_TT, _C, _H),
                     acnt_ref[...], bias_ref[...], _TT, _C, _T)
    zp = jnp.zeros((_CP - _C, _G), jnp.float32)
    out_ref[0] = jnp.concatenate(
        [jnp.concatenate([g, zp], axis=0)[:, None, :] for g in outs], axis=1)


def _temporal_kernel(gs_ref, acnt_ref, wt_ref, mst_ref, mdt_ref, bias_ref,
                     out_ref):
    xin = gs_ref[0]
    xp = jnp.dot(xin.reshape(_CCT * _T, _G), wt_ref[...],
                 preferred_element_type=jnp.float32)
    asm = jnp.dot(xp, mst_ref[...], preferred_element_type=jnp.float32)
    ad = jnp.dot(xp, mdt_ref[...], preferred_element_type=jnp.float32)
    xp3 = xp.reshape(_CCT, _T, _G)
    outs = _gat_tile(xp3, asm.reshape(_CCT, _T, _H), ad.reshape(_CCT, _T, _H),
                     acnt_ref[...], bias_ref[...], _CCT, _T, _T)
    out_ref[0] = jnp.concatenate(outs, axis=1)


def _shift_rows(x, s):
    if s == 0:
        return x
    return jnp.concatenate(
        [jnp.zeros((s, x.shape[1]), x.dtype), x[:-s]], axis=0)


def _causal_conv(xin, w_ref, b, d):
    acc = jnp.dot(_shift_rows(xin, 2 * d), w_ref[0],
                  preferred_element_type=jnp.float32)
    acc = acc + jnp.dot(_shift_rows(xin, d), w_ref[1],
                        preferred_element_type=jnp.float32)
    acc = acc + jnp.dot(xin, w_ref[2], preferred_element_type=jnp.float32)
    return acc + b


def _tcn_kernel(x_ref, w1a_ref, w1b_ref, dw_ref, w2a_ref, w2b_ref,
                b1a_ref, b1b_ref, db_ref, g1_ref, be1_ref, m1_ref, v1_ref,
                b2a_ref, b2b_ref, g2_ref, be2_ref, m2_ref, v2_ref, out_ref):
    x = x_ref[0]
    res = jnp.dot(x, dw_ref[...], preferred_element_type=jnp.float32) \
        + db_ref[...]
    h = jax.nn.relu(_causal_conv(x, w1a_ref, b1a_ref[...], 1))
    h = jax.nn.relu(_causal_conv(h, w1b_ref, b1b_ref[...], 1))
    h = h + res
    scale1 = g1_ref[...] * jax.lax.rsqrt(v1_ref[...] + 1e-5)
    h = (h - m1_ref[...]) * scale1 + be1_ref[...]
    res2 = h
    h = jax.nn.relu(_causal_conv(h, w2a_ref, b2a_ref[...], 2))
    h = jax.nn.relu(_causal_conv(h, w2b_ref, b2b_ref[...], 2))
    h = h + res2
    scale2 = g2_ref[...] * jax.lax.rsqrt(v2_ref[...] + 1e-5)
    h = (h - m2_ref[...]) * scale2 + be2_ref[...]
    out_ref[0] = jnp.max(h, axis=0, keepdims=True)


def kernel(x, spatial_edge_index, temporal_edge_index, W_spec, b_spec, Ws,
           asrc_s, adst_s, bias_s, Wt, asrc_t, adst_t, bias_t,
           tb1_w1, tb1_b1, tb1_w2, tb1_b2, tb1_dw, tb1_db,
           tb1_gamma, tb1_beta, tb1_mean, tb1_var,
           tb2_w1, tb2_b1, tb2_w2, tb2_b2,
           tb2_gamma, tb2_beta, tb2_mean, tb2_var):
    f32 = jnp.float32
    idt = spatial_edge_index.dtype

    sl_c = jnp.arange(_C, dtype=idt)
    sl_t = jnp.arange(_T, dtype=idt)
    pad_s = jnp.full((_ESP - _ES - _C,), -1, idt)
    src_s = jnp.concatenate([spatial_edge_index[0], sl_c, pad_s])
    dst_s = jnp.concatenate([spatial_edge_index[1], sl_c, pad_s])
    pad_t = jnp.full((_ETP - _ET - _T,), -1, idt)
    src_t = jnp.concatenate([temporal_edge_index[0], sl_t, pad_t])
    dst_t = jnp.concatenate([temporal_edge_index[1], sl_t, pad_t])

    a_s, a_t, wf, bf, mss, mst, mds, mdt = pl.pallas_call(
        _prep_kernel,
        out_shape=(
            jax.ShapeDtypeStruct((_C, _T), f32),
            jax.ShapeDtypeStruct((_T, _T), f32),
            jax.ShapeDtypeStruct((_F, _G), f32),
            jax.ShapeDtypeStruct((1, _G), f32),
            jax.ShapeDtypeStruct((_G, _H), f32),
            jax.ShapeDtypeStruct((_G, _H), f32),
            jax.ShapeDtypeStruct((_G, _H), f32),
            jax.ShapeDtypeStruct((_G, _H), f32),
        ),
    )(src_s.reshape(_ESP, 1), dst_s.reshape(1, _ESP),
      src_t.reshape(_ETP, 1), dst_t.reshape(1, _ETP),
      W_spec, Ws, b_spec.reshape(1, _DS),
      asrc_s.reshape(_G, 1), asrc_t.reshape(_G, 1),
      adst_s.reshape(_G, 1), adst_t.reshape(_G, 1))

    xT = jnp.transpose(x, (0, 3, 2, 1))
    gs = pl.pallas_call(
        _spatial_kernel,
        grid=(_B, _T // _TT),
        in_specs=[
            pl.BlockSpec((1, _TT, _C, _F), lambda b, t: (b, t, 0, 0)),
            pl.BlockSpec((_C, _T), lambda b, t: (0, 0)),
            pl.BlockSpec((_F, _G), lambda b, t: (0, 0)),
            pl.BlockSpec((1, _G), lambda b, t: (0, 0)),
            pl.BlockSpec((_G, _H), lambda b, t: (0, 0)),
            pl.BlockSpec((_G, _H), lambda b, t: (0, 0)),
            pl.BlockSpec((1, _G), lambda b, t: (0, 0)),
        ],
        out_specs=pl.BlockSpec((1, _CP, _TT, _G), lambda b, t: (b, 0, t, 0)),
        out_shape=jax.ShapeDtypeStruct((_B, _CP, _T, _G), f32),
    )(xT, a_s, wf, bf, mss, mds, bias_s.reshape(1, _G))

    tcnin = pl.pallas_call(
        _temporal_kernel,
        grid=(_B, _CP // _CCT),
        in_specs=[
            pl.BlockSpec((1, _CCT, _T, _G), lambda b, c: (b, c, 0, 0)),
            pl.BlockSpec((_T, _T), lambda b, c: (0, 0)),
            pl.BlockSpec((_G, _G), lambda b, c: (0, 0)),
            pl.BlockSpec((_G, _H), lambda b, c: (0, 0)),
            pl.BlockSpec((_G, _H), lambda b, c: (0, 0)),
            pl.BlockSpec((1, _G), lambda b, c: (0, 0)),
        ],
        out_specs=pl.BlockSpec((1, _T, _CCT * _G), lambda b, c: (b, 0, c)),
        out_shape=jax.ShapeDtypeStruct((_B, _T, _CHP), f32),
    )(gs, a_t, Wt, mst, mdt, bias_t.reshape(1, _G))

    zpad = jnp.zeros((3, _CHP - _CH, _O), f32)
    w1a = jnp.concatenate([jnp.transpose(tb1_w1, (2, 1, 0)), zpad], axis=1)
    w1b = jnp.transpose(tb1_w2, (2, 1, 0))
    dw = jnp.concatenate(
        [jnp.transpose(tb1_dw[:, :, 0], (1, 0)),
         jnp.zeros((_CHP - _CH, _O), f32)], axis=0)
    w2a = jnp.transpose(tb2_w1, (2, 1, 0))
    w2b = jnp.transpose(tb2_w2, (2, 1, 0))

    def row(v):
        return v.reshape(1, _O)

    def full(*shape):
        return [pl.BlockSpec(shape, lambda b: tuple(0 for _ in shape))]

    out = pl.pallas_call(
        _tcn_kernel,
        grid=(_B,),
        in_specs=(
            [pl.BlockSpec((1, _T, _CHP), lambda b: (b, 0, 0))]
            + full(3, _CHP, _O) + full(3, _O, _O) + full(_CHP, _O)
            + full(3, _O, _O) + full(3, _O, _O)
            + full(1, _O) * 13
        ),
        out_specs=pl.BlockSpec((1, 1, _O), lambda b: (b, 0, 0)),
        out_shape=jax.ShapeDtypeStruct((_B, 1, _O), f32),
    )(tcnin, w1a, w1b, dw, w2a, w2b,
      row(tb1_b1), row(tb1_b2), row(tb1_db),
      row(tb1_gamma), row(tb1_beta), row(tb1_mean), row(tb1_var),
      row(tb2_b1), row(tb2_b2),
      row(tb2_gamma), row(tb2_beta), row(tb2_mean), row(tb2_var))

    return out.reshape(_B, _O)


# speedup vs baseline: 6.2614x; 1.4154x over previous
import jax
import jax.numpy as jnp
from jax.experimental import pallas as pl

_B = 16
_F = 5
_DS = 64
_C = 62
_T = 128
_G = 32
_H = 4
_DH = 8
_CP = 64
_TT = 8
_CCT = 4
_ES = 512
_ET = 512
_ESP = 576
_ETP = 640
_O = 128
_CH = _C * _G
_CHP = _CP * _G


def _leaky(x):
    return jnp.where(x >= 0, x, 0.2 * x)


def _prep_kernel(srcs_ref, dsts_ref, srct_ref, dstt_ref, wspec_ref, ws_ref,
                 bspec_ref, asrcs_ref, asrct_ref, adsts_ref, adstt_ref,
                 a_s_ref, a_t_ref, wf_ref, bf_ref, mss_ref, mst_ref,
                 mds_ref, mdt_ref):
    dh_s = (jax.lax.broadcasted_iota(jnp.int32, (_CP, _ESP), 0)
            == dsts_ref[...]).astype(jnp.float32)
    sh_s = (jax.lax.broadcasted_iota(jnp.int32, (_ESP, _T), 1)
            == srcs_ref[...]).astype(jnp.float32)
    a_s_ref[...] = jnp.dot(dh_s, sh_s, preferred_element_type=jnp.float32)
    dh_t = (jax.lax.broadcasted_iota(jnp.int32, (_T, _ETP), 0)
            == dstt_ref[...]).astype(jnp.float32)
    sh_t = (jax.lax.broadcasted_iota(jnp.int32, (_ETP, _T), 1)
            == srct_ref[...]).astype(jnp.float32)
    a_t_ref[...] = jnp.dot(dh_t, sh_t, preferred_element_type=jnp.float32)
    wf_ref[...] = jnp.dot(wspec_ref[...], ws_ref[...],
                          preferred_element_type=jnp.float32)
    bf_ref[...] = jnp.dot(bspec_ref[...], ws_ref[...],
                          preferred_element_type=jnp.float32)
    rows = jax.lax.broadcasted_iota(jnp.int32, (_G, _H), 0)
    cols = jax.lax.broadcasted_iota(jnp.int32, (_G, _H), 1)
    blk = (rows // _DH == cols).astype(jnp.float32)
    mss_ref[...] = asrcs_ref[...] * blk
    mst_ref[...] = asrct_ref[...] * blk
    mds_ref[...] = adsts_ref[...] * blk
    mdt_ref[...] = adstt_ref[...] * blk


def _hmasks():
    return [(jax.lax.broadcasted_iota(jnp.int32, (1, _G), 1) // _DH == h
             ).astype(jnp.float32) for h in range(_H)]


def _gat_tile(xp3, asm3, ad3, acnt, bias, nb, n, npad):
    asT = jnp.transpose(asm3, (0, 2, 1))
    if npad != n:
        asT = jnp.concatenate(
            [asT, jnp.zeros((nb, _H, npad - n), jnp.float32)], axis=2)
    nums, dens = [], []
    for h in range(_H):
        arow = asT[:, h:h + 1, :]
        adh = ad3[:, :, h:h + 1]
        pos = (adh + arow) >= 0
        seld = jnp.where(pos, jnp.exp(adh), jnp.exp(0.2 * adh))
        sels = jnp.where(pos, jnp.exp(arow), jnp.exp(0.2 * arow))
        num = acnt * seld * sels
        nums.append(num)
        dens.append(jnp.sum(num, axis=2, keepdims=True))
    recc = 1.0 / (jnp.concatenate(dens, axis=2) + 1e-16)
    hmask = _hmasks()
    r8 = jnp.concatenate(hmask, axis=0)
    outs = []
    for b in range(nb):
        if npad == n:
            xpb = xp3[b]
        else:
            xpb = jnp.concatenate(
                [xp3[b], jnp.zeros((npad - n, _G), jnp.float32)], axis=0)
        o = None
        for h in range(_H):
            oh = jnp.dot(nums[h][b], xpb,
                         preferred_element_type=jnp.float32) * hmask[h]
            o = oh if o is None else o + oh
        r = jnp.dot(recc[b], r8, preferred_element_type=jnp.float32)
        outs.append(_leaky(o * r + bias))
    return outs


def _spatial_kernel(x_ref, acnt_ref, wf_ref, bf_ref, mss_ref, mds_ref,
                    bias_ref, out_ref):
    xb = x_ref[0]
    xp = jnp.dot(xb.reshape(_TT * _CP, _F), wf_ref[...],
                 preferred_element_type=jnp.float32) + bf_ref[...]
    asm = jnp.dot(xp, mss_ref[...], preferred_element_type=jnp.float32)
    ad = jnp.dot(xp, mds_ref[...], preferred_element_type=jnp.float32)
    xp3 = xp.reshape(_TT, _CP, _G)
    outs = _gat_tile(xp3, asm.reshape(_TT, _CP, _H), ad.reshape(_TT, _CP, _H),
                     acnt_ref[...], bias_ref[...], _TT, _CP, _T)
    out_ref[0] = jnp.concatenate([g[:, None, :] for g in outs], axis=1)


def _temporal_kernel(gs_ref, acnt_ref, wt_ref, mst_ref, mdt_ref, bias_ref,
                     out_ref):
    xin = gs_ref[0]
    xp = jnp.dot(xin.reshape(_CCT * _T, _G), wt_ref[...],
                 preferred_element_type=jnp.float32)
    asm = jnp.dot(xp, mst_ref[...], preferred_element_type=jnp.float32)
    ad = jnp.dot(xp, mdt_ref[...], preferred_element_type=jnp.float32)
    xp3 = xp.reshape(_CCT, _T, _G)
    outs = _gat_tile(xp3, asm.reshape(_CCT, _T, _H), ad.reshape(_CCT, _T, _H),
                     acnt_ref[...], bias_ref[...], _CCT, _T, _T)
    out_ref[0] = jnp.concatenate(outs, axis=1)


def _shift_rows(x, s):
    if s == 0:
        return x
    return jnp.concatenate(
        [jnp.zeros((s, x.shape[1]), x.dtype), x[:-s]], axis=0)


def _causal_conv(xin, w_ref, b, d):
    acc = jnp.dot(_shift_rows(xin, 2 * d), w_ref[0],
                  preferred_element_type=jnp.float32)
    acc = acc + jnp.dot(_shift_rows(xin, d), w_ref[1],
                        preferred_element_type=jnp.float32)
    acc = acc + jnp.dot(xin, w_ref[2], preferred_element_type=jnp.float32)
    return acc + b


def _tcn_kernel(x_ref, w1a_ref, w1b_ref, dw_ref, w2a_ref, w2b_ref,
                b1a_ref, b1b_ref, db_ref, g1_ref, be1_ref, m1_ref, v1_ref,
                b2a_ref, b2b_ref, g2_ref, be2_ref, m2_ref, v2_ref, out_ref):
    x = x_ref[0]
    res = jnp.dot(x, dw_ref[...], preferred_element_type=jnp.float32) \
        + db_ref[...]
    h = jax.nn.relu(_causal_conv(x, w1a_ref, b1a_ref[...], 1))
    h = jax.nn.relu(_causal_conv(h, w1b_ref, b1b_ref[...], 1))
    h = h + res
    scale1 = g1_ref[...] * jax.lax.rsqrt(v1_ref[...] + 1e-5)
    h = (h - m1_ref[...]) * scale1 + be1_ref[...]
    res2 = h
    h = jax.nn.relu(_causal_conv(h, w2a_ref, b2a_ref[...], 2))
    h = jax.nn.relu(_causal_conv(h, w2b_ref, b2b_ref[...], 2))
    h = h + res2
    scale2 = g2_ref[...] * jax.lax.rsqrt(v2_ref[...] + 1e-5)
    h = (h - m2_ref[...]) * scale2 + be2_ref[...]
    out_ref[0] = jnp.max(h, axis=0, keepdims=True)


def kernel(x, spatial_edge_index, temporal_edge_index, W_spec, b_spec, Ws,
           asrc_s, adst_s, bias_s, Wt, asrc_t, adst_t, bias_t,
           tb1_w1, tb1_b1, tb1_w2, tb1_b2, tb1_dw, tb1_db,
           tb1_gamma, tb1_beta, tb1_mean, tb1_var,
           tb2_w1, tb2_b1, tb2_w2, tb2_b2,
           tb2_gamma, tb2_beta, tb2_mean, tb2_var):
    f32 = jnp.float32
    idt = spatial_edge_index.dtype

    sl_c = jnp.arange(_C, dtype=idt)
    sl_t = jnp.arange(_T, dtype=idt)
    pad_s = jnp.full((_ESP - _ES - _C,), -1, idt)
    src_s = jnp.concatenate([spatial_edge_index[0], sl_c, pad_s])
    dst_s = jnp.concatenate([spatial_edge_index[1], sl_c, pad_s])
    pad_t = jnp.full((_ETP - _ET - _T,), -1, idt)
    src_t = jnp.concatenate([temporal_edge_index[0], sl_t, pad_t])
    dst_t = jnp.concatenate([temporal_edge_index[1], sl_t, pad_t])

    a_s, a_t, wf, bf, mss, mst, mds, mdt = pl.pallas_call(
        _prep_kernel,
        out_shape=(
            jax.ShapeDtypeStruct((_CP, _T), f32),
            jax.ShapeDtypeStruct((_T, _T), f32),
            jax.ShapeDtypeStruct((_F, _G), f32),
            jax.ShapeDtypeStruct((1, _G), f32),
            jax.ShapeDtypeStruct((_G, _H), f32),
            jax.ShapeDtypeStruct((_G, _H), f32),
            jax.ShapeDtypeStruct((_G, _H), f32),
            jax.ShapeDtypeStruct((_G, _H), f32),
        ),
    )(src_s.reshape(_ESP, 1), dst_s.reshape(1, _ESP),
      src_t.reshape(_ETP, 1), dst_t.reshape(1, _ETP),
      W_spec, Ws, b_spec.reshape(1, _DS),
      asrc_s.reshape(_G, 1), asrc_t.reshape(_G, 1),
      adst_s.reshape(_G, 1), adst_t.reshape(_G, 1))

    xT = jnp.pad(jnp.transpose(x, (0, 3, 2, 1)),
                 ((0, 0), (0, 0), (0, _CP - _C), (0, 0)))
    gs = pl.pallas_call(
        _spatial_kernel,
        grid=(_B, _T // _TT),
        in_specs=[
            pl.BlockSpec((1, _TT, _CP, _F), lambda b, t: (b, t, 0, 0)),
            pl.BlockSpec((_CP, _T), lambda b, t: (0, 0)),
            pl.BlockSpec((_F, _G), lambda b, t: (0, 0)),
            pl.BlockSpec((1, _G), lambda b, t: (0, 0)),
            pl.BlockSpec((_G, _H), lambda b, t: (0, 0)),
            pl.BlockSpec((_G, _H), lambda b, t: (0, 0)),
            pl.BlockSpec((1, _G), lambda b, t: (0, 0)),
        ],
        out_specs=pl.BlockSpec((1, _CP, _TT, _G), lambda b, t: (b, 0, t, 0)),
        out_shape=jax.ShapeDtypeStruct((_B, _CP, _T, _G), f32),
    )(xT, a_s, wf, bf, mss, mds, bias_s.reshape(1, _G))

    tcnin = pl.pallas_call(
        _temporal_kernel,
        grid=(_B, _CP // _CCT),
        in_specs=[
            pl.BlockSpec((1, _CCT, _T, _G), lambda b, c: (b, c, 0, 0)),
            pl.BlockSpec((_T, _T), lambda b, c: (0, 0)),
            pl.BlockSpec((_G, _G), lambda b, c: (0, 0)),
            pl.BlockSpec((_G, _H), lambda b, c: (0, 0)),
            pl.BlockSpec((_G, _H), lambda b, c: (0, 0)),
            pl.BlockSpec((1, _G), lambda b, c: (0, 0)),
        ],
        out_specs=pl.BlockSpec((1, _T, _CCT * _G), lambda b, c: (b, 0, c)),
        out_shape=jax.ShapeDtypeStruct((_B, _T, _CHP), f32),
    )(gs, a_t, Wt, mst, mdt, bias_t.reshape(1, _G))

    zpad = jnp.zeros((3, _CHP - _CH, _O), f32)
    w1a = jnp.concatenate([jnp.transpose(tb1_w1, (2, 1, 0)), zpad], axis=1)
    w1b = jnp.transpose(tb1_w2, (2, 1, 0))
    dw = jnp.concatenate(
        [jnp.transpose(tb1_dw[:, :, 0], (1, 0)),
         jnp.zeros((_CHP - _CH, _O), f32)], axis=0)
    w2a = jnp.transpose(tb2_w1, (2, 1, 0))
    w2b = jnp.transpose(tb2_w2, (2, 1, 0))

    def row(v):
        return v.reshape(1, _O)

    def full(*shape):
        return [pl.BlockSpec(shape, lambda b: tuple(0 for _ in shape))]

    out = pl.pallas_call(
        _tcn_kernel,
        grid=(_B,),
        in_specs=(
            [pl.BlockSpec((1, _T, _CHP), lambda b: (b, 0, 0))]
            + full(3, _CHP, _O) + full(3, _O, _O) + full(_CHP, _O)
            + full(3, _O, _O) + full(3, _O, _O)
            + full(1, _O) * 13
        ),
        out_specs=pl.BlockSpec((1, 1, _O), lambda b: (b, 0, 0)),
        out_shape=jax.ShapeDtypeStruct((_B, 1, _O), f32),
    )(tcnin, w1a, w1b, dw, w2a, w2b,
      row(tb1_b1), row(tb1_b2), row(tb1_db),
      row(tb1_gamma), row(tb1_beta), row(tb1_mean), row(tb1_var),
      row(tb2_b1), row(tb2_b2),
      row(tb2_gamma), row(tb2_beta), row(tb2_mean), row(tb2_var))

    return out.reshape(_B, _O)


# speedup vs baseline: 7.4873x; 1.1958x over previous
import jax
import jax.numpy as jnp
from jax.experimental import pallas as pl

_B = 16
_F = 5
_DS = 64
_C = 62
_T = 128
_G = 32
_H = 4
_DH = 8
_CP = 64
_TT = 16
_CCT = 8
_ES = 512
_ET = 512
_ESP = 576
_ETP = 640
_O = 128
_CH = _C * _G
_CHP = _CP * _G


def _leaky(x):
    return jnp.where(x >= 0, x, 0.2 * x)


def _prep_kernel(srcs_ref, dsts_ref, srct_ref, dstt_ref, wspec_ref, ws_ref,
                 bspec_ref, asrcs_ref, asrct_ref, adsts_ref, adstt_ref,
                 a_s_ref, a_t_ref, wf_ref, bf_ref, mss_ref, mst_ref,
                 mds_ref, mdt_ref):
    dh_s = (jax.lax.broadcasted_iota(jnp.int32, (_CP, _ESP), 0)
            == dsts_ref[...]).astype(jnp.float32)
    sh_s = (jax.lax.broadcasted_iota(jnp.int32, (_ESP, _T), 1)
            == srcs_ref[...]).astype(jnp.float32)
    a_s_ref[...] = jnp.dot(dh_s, sh_s, preferred_element_type=jnp.float32)
    dh_t = (jax.lax.broadcasted_iota(jnp.int32, (_T, _ETP), 0)
            == dstt_ref[...]).astype(jnp.float32)
    sh_t = (jax.lax.broadcasted_iota(jnp.int32, (_ETP, _T), 1)
            == srct_ref[...]).astype(jnp.float32)
    a_t_ref[...] = jnp.dot(dh_t, sh_t, preferred_element_type=jnp.float32)
    wf_ref[...] = jnp.dot(wspec_ref[...], ws_ref[...],
                          preferred_element_type=jnp.float32)
    bf_ref[...] = jnp.dot(bspec_ref[...], ws_ref[...],
                          preferred_element_type=jnp.float32)
    rows = jax.lax.broadcasted_iota(jnp.int32, (_G, _H), 0)
    cols = jax.lax.broadcasted_iota(jnp.int32, (_G, _H), 1)
    blk = (rows // _DH == cols).astype(jnp.float32)
    mss_ref[...] = asrcs_ref[...] * blk
    mst_ref[...] = asrct_ref[...] * blk
    mds_ref[...] = adsts_ref[...] * blk
    mdt_ref[...] = adstt_ref[...] * blk


def _hmasks():
    return [(jax.lax.broadcasted_iota(jnp.int32, (1, _G), 1) // _DH == h
             ).astype(jnp.float32) for h in range(_H)]


def _gat_tile(xp3, asm3, ad3, acnt, bias, nb, n, npad):
    asT = jnp.transpose(asm3, (0, 2, 1))
    if npad != n:
        asT = jnp.concatenate(
            [asT, jnp.zeros((nb, _H, npad - n), jnp.float32)], axis=2)
    nums, dens = [], []
    for h in range(_H):
        arow = asT[:, h:h + 1, :]
        adh = ad3[:, :, h:h + 1]
        pos = (adh + arow) >= 0
        seld = jnp.where(pos, jnp.exp(adh), jnp.exp(0.2 * adh))
        sels = jnp.where(pos, jnp.exp(arow), jnp.exp(0.2 * arow))
        num = acnt * seld * sels
        nums.append(num)
        dens.append(jnp.sum(num, axis=2, keepdims=True))
    recc = 1.0 / (jnp.concatenate(dens, axis=2) + 1e-16)
    hmask = _hmasks()
    r8 = jnp.concatenate(hmask, axis=0)
    outs = []
    for b in range(nb):
        if npad == n:
            xpb = xp3[b]
        else:
            xpb = jnp.concatenate(
                [xp3[b], jnp.zeros((npad - n, _G), jnp.float32)], axis=0)
        o = None
        for h in range(_H):
            oh = jnp.dot(nums[h][b], xpb,
                         preferred_element_type=jnp.float32) * hmask[h]
            o = oh if o is None else o + oh
        r = jnp.dot(recc[b], r8, preferred_element_type=jnp.float32)
        outs.append(_leaky(o * r + bias))
    return outs


def _spatial_kernel(x_ref, acnt_ref, wf_ref, bf_ref, mss_ref, mds_ref,
                    bias_ref, out_ref):
    xb = x_ref[0]
    xp = jnp.dot(xb.reshape(_TT * _CP, _F), wf_ref[...],
                 preferred_element_type=jnp.float32) + bf_ref[...]
    asm = jnp.dot(xp, mss_ref[...], preferred_element_type=jnp.float32)
    ad = jnp.dot(xp, mds_ref[...], preferred_element_type=jnp.float32)
    xp3 = xp.reshape(_TT, _CP, _G)
    outs = _gat_tile(xp3, asm.reshape(_TT, _CP, _H), ad.reshape(_TT, _CP, _H),
                     acnt_ref[...], bias_ref[...], _TT, _CP, _T)
    out_ref[0] = jnp.concatenate([g[:, None, :] for g in outs], axis=1)


def _temporal_kernel(gs_ref, acnt_ref, wt_ref, mst_ref, mdt_ref, bias_ref,
                     out_ref):
    xin = gs_ref[0]
    xp = jnp.dot(xin.reshape(_CCT * _T, _G), wt_ref[...],
                 preferred_element_type=jnp.float32)
    asm = jnp.dot(xp, mst_ref[...], preferred_element_type=jnp.float32)
    ad = jnp.dot(xp, mdt_ref[...], preferred_element_type=jnp.float32)
    xp3 = xp.reshape(_CCT, _T, _G)
    outs = _gat_tile(xp3, asm.reshape(_CCT, _T, _H), ad.reshape(_CCT, _T, _H),
                     acnt_ref[...], bias_ref[...], _CCT, _T, _T)
    out_ref[0] = jnp.concatenate(outs, axis=1)


def _shift_rows(x, s):
    if s == 0:
        return x
    return jnp.concatenate(
        [jnp.zeros((s, x.shape[1]), x.dtype), x[:-s]], axis=0)


def _causal_conv(xin, w_ref, b, d):
    acc = jnp.dot(_shift_rows(xin, 2 * d), w_ref[0],
                  preferred_element_type=jnp.float32)
    acc = acc + jnp.dot(_shift_rows(xin, d), w_ref[1],
                        preferred_element_type=jnp.float32)
    acc = acc + jnp.dot(xin, w_ref[2], preferred_element_type=jnp.float32)
    return acc + b


def _tcn_kernel(x_ref, w1a_ref, w1b_ref, dw_ref, w2a_ref, w2b_ref,
                b1a_ref, b1b_ref, db_ref, g1_ref, be1_ref, m1_ref, v1_ref,
                b2a_ref, b2b_ref, g2_ref, be2_ref, m2_ref, v2_ref, out_ref):
    x = x_ref[0]
    res = jnp.dot(x, dw_ref[...], preferred_element_type=jnp.float32) \
        + db_ref[...]
    h = jax.nn.relu(_causal_conv(x, w1a_ref, b1a_ref[...], 1))
    h = jax.nn.relu(_causal_conv(h, w1b_ref, b1b_ref[...], 1))
    h = h + res
    scale1 = g1_ref[...] * jax.lax.rsqrt(v1_ref[...] + 1e-5)
    h = (h - m1_ref[...]) * scale1 + be1_ref[...]
    res2 = h
    h = jax.nn.relu(_causal_conv(h, w2a_ref, b2a_ref[...], 2))
    h = jax.nn.relu(_causal_conv(h, w2b_ref, b2b_ref[...], 2))
    h = h + res2
    scale2 = g2_ref[...] * jax.lax.rsqrt(v2_ref[...] + 1e-5)
    h = (h - m2_ref[...]) * scale2 + be2_ref[...]
    out_ref[0] = jnp.max(h, axis=0, keepdims=True)


def kernel(x, spatial_edge_index, temporal_edge_index, W_spec, b_spec, Ws,
           asrc_s, adst_s, bias_s, Wt, asrc_t, adst_t, bias_t,
           tb1_w1, tb1_b1, tb1_w2, tb1_b2, tb1_dw, tb1_db,
           tb1_gamma, tb1_beta, tb1_mean, tb1_var,
           tb2_w1, tb2_b1, tb2_w2, tb2_b2,
           tb2_gamma, tb2_beta, tb2_mean, tb2_var):
    f32 = jnp.float32
    idt = spatial_edge_index.dtype

    sl_c = jnp.arange(_C, dtype=idt)
    sl_t = jnp.arange(_T, dtype=idt)
    pad_s = jnp.full((_ESP - _ES - _C,), -1, idt)
    src_s = jnp.concatenate([spatial_edge_index[0], sl_c, pad_s])
    dst_s = jnp.concatenate([spatial_edge_index[1], sl_c, pad_s])
    pad_t = jnp.full((_ETP - _ET - _T,), -1, idt)
    src_t = jnp.concatenate([temporal_edge_index[0], sl_t, pad_t])
    dst_t = jnp.concatenate([temporal_edge_index[1], sl_t, pad_t])

    a_s, a_t, wf, bf, mss, mst, mds, mdt = pl.pallas_call(
        _prep_kernel,
        out_shape=(
            jax.ShapeDtypeStruct((_CP, _T), f32),
            jax.ShapeDtypeStruct((_T, _T), f32),
            jax.ShapeDtypeStruct((_F, _G), f32),
            jax.ShapeDtypeStruct((1, _G), f32),
            jax.ShapeDtypeStruct((_G, _H), f32),
            jax.ShapeDtypeStruct((_G, _H), f32),
            jax.ShapeDtypeStruct((_G, _H), f32),
            jax.ShapeDtypeStruct((_G, _H), f32),
        ),
    )(src_s.reshape(_ESP, 1), dst_s.reshape(1, _ESP),
      src_t.reshape(_ETP, 1), dst_t.reshape(1, _ETP),
      W_spec, Ws, b_spec.reshape(1, _DS),
      asrc_s.reshape(_G, 1), asrc_t.reshape(_G, 1),
      adst_s.reshape(_G, 1), adst_t.reshape(_G, 1))

    xT = jnp.pad(jnp.transpose(x, (0, 3, 2, 1)),
                 ((0, 0), (0, 0), (0, _CP - _C), (0, 0)))
    gs = pl.pallas_call(
        _spatial_kernel,
        grid=(_B, _T // _TT),
        in_specs=[
            pl.BlockSpec((1, _TT, _CP, _F), lambda b, t: (b, t, 0, 0)),
            pl.BlockSpec((_CP, _T), lambda b, t: (0, 0)),
            pl.BlockSpec((_F, _G), lambda b, t: (0, 0)),
            pl.BlockSpec((1, _G), lambda b, t: (0, 0)),
            pl.BlockSpec((_G, _H), lambda b, t: (0, 0)),
            pl.BlockSpec((_G, _H), lambda b, t: (0, 0)),
            pl.BlockSpec((1, _G), lambda b, t: (0, 0)),
        ],
        out_specs=pl.BlockSpec((1, _CP, _TT, _G), lambda b, t: (b, 0, t, 0)),
        out_shape=jax.ShapeDtypeStruct((_B, _CP, _T, _G), f32),
    )(xT, a_s, wf, bf, mss, mds, bias_s.reshape(1, _G))

    tcnin = pl.pallas_call(
        _temporal_kernel,
        grid=(_B, _CP // _CCT),
        in_specs=[
            pl.BlockSpec((1, _CCT, _T, _G), lambda b, c: (b, c, 0, 0)),
            pl.BlockSpec((_T, _T), lambda b, c: (0, 0)),
            pl.BlockSpec((_G, _G), lambda b, c: (0, 0)),
            pl.BlockSpec((_G, _H), lambda b, c: (0, 0)),
            pl.BlockSpec((_G, _H), lambda b, c: (0, 0)),
            pl.BlockSpec((1, _G), lambda b, c: (0, 0)),
        ],
        out_specs=pl.BlockSpec((1, _T, _CCT * _G), lambda b, c: (b, 0, c)),
        out_shape=jax.ShapeDtypeStruct((_B, _T, _CHP), f32),
    )(gs, a_t, Wt, mst, mdt, bias_t.reshape(1, _G))

    zpad = jnp.zeros((3, _CHP - _CH, _O), f32)
    w1a = jnp.concatenate([jnp.transpose(tb1_w1, (2, 1, 0)), zpad], axis=1)
    w1b = jnp.transpose(tb1_w2, (2, 1, 0))
    dw = jnp.concatenate(
        [jnp.transpose(tb1_dw[:, :, 0], (1, 0)),
         jnp.zeros((_CHP - _CH, _O), f32)], axis=0)
    w2a = jnp.transpose(tb2_w1, (2, 1, 0))
    w2b = jnp.transpose(tb2_w2, (2, 1, 0))

    def row(v):
        return v.reshape(1, _O)

    def full(*shape):
        return [pl.BlockSpec(shape, lambda b: tuple(0 for _ in shape))]

    out = pl.pallas_call(
        _tcn_kernel,
        grid=(_B,),
        in_specs=(
            [pl.BlockSpec((1, _T, _CHP), lambda b: (b, 0, 0))]
            + full(3, _CHP, _O) + full(3, _O, _O) + full(_CHP, _O)
            + full(3, _O, _O) + full(3, _O, _O)
            + full(1, _O) * 13
        ),
        out_specs=pl.BlockSpec((1, 1, _O), lambda b: (b, 0, 0)),
        out_shape=jax.ShapeDtypeStruct((_B, 1, _O), f32),
    )(tcnin, w1a, w1b, dw, w2a, w2b,
      row(tb1_b1), row(tb1_b2), row(tb1_db),
      row(tb1_gamma), row(tb1_beta), row(tb1_mean), row(tb1_var),
      row(tb2_b1), row(tb2_b2),
      row(tb2_gamma), row(tb2_beta), row(tb2_mean), row(tb2_var))

    return out.reshape(_B, _O)


# speedup vs baseline: 8.4845x; 1.1332x over previous
import jax
import jax.numpy as jnp
from jax.experimental import pallas as pl

_B = 16
_F = 5
_DS = 64
_C = 62
_T = 128
_G = 32
_H = 4
_DH = 8
_CP = 64
_TT = 32
_CCT = 16
_ES = 512
_ET = 512
_ESP = 576
_ETP = 640
_O = 128
_CH = _C * _G
_CHP = _CP * _G


def _leaky(x):
    return jnp.where(x >= 0, x, 0.2 * x)


def _prep_kernel(srcs_ref, dsts_ref, srct_ref, dstt_ref, wspec_ref, ws_ref,
                 bspec_ref, asrcs_ref, asrct_ref, adsts_ref, adstt_ref,
                 a_s_ref, a_t_ref, wf_ref, bf_ref, mss_ref, mst_ref,
                 mds_ref, mdt_ref):
    dh_s = (jax.lax.broadcasted_iota(jnp.int32, (_CP, _ESP), 0)
            == dsts_ref[...]).astype(jnp.float32)
    sh_s = (jax.lax.broadcasted_iota(jnp.int32, (_ESP, _T), 1)
            == srcs_ref[...]).astype(jnp.float32)
    a_s_ref[...] = jnp.dot(dh_s, sh_s, preferred_element_type=jnp.float32)
    dh_t = (jax.lax.broadcasted_iota(jnp.int32, (_T, _ETP), 0)
            == dstt_ref[...]).astype(jnp.float32)
    sh_t = (jax.lax.broadcasted_iota(jnp.int32, (_ETP, _T), 1)
            == srct_ref[...]).astype(jnp.float32)
    a_t_ref[...] = jnp.dot(dh_t, sh_t, preferred_element_type=jnp.float32)
    wf_ref[...] = jnp.dot(wspec_ref[...], ws_ref[...],
                          preferred_element_type=jnp.float32)
    bf_ref[...] = jnp.dot(bspec_ref[...], ws_ref[...],
                          preferred_element_type=jnp.float32)
    rows = jax.lax.broadcasted_iota(jnp.int32, (_G, _H), 0)
    cols = jax.lax.broadcasted_iota(jnp.int32, (_G, _H), 1)
    blk = (rows // _DH == cols).astype(jnp.float32)
    mss_ref[...] = asrcs_ref[...] * blk
    mst_ref[...] = asrct_ref[...] * blk
    mds_ref[...] = adsts_ref[...] * blk
    mdt_ref[...] = adstt_ref[...] * blk


def _hmasks():
    return [(jax.lax.broadcasted_iota(jnp.int32, (1, _G), 1) // _DH == h
             ).astype(jnp.float32) for h in range(_H)]


def _gat_tile(xp3, asm3, ad3, acnt, bias, nb, n, npad):
    asT = jnp.transpose(asm3, (0, 2, 1))
    if npad != n:
        asT = jnp.concatenate(
            [asT, jnp.zeros((nb, _H, npad - n), jnp.float32)], axis=2)
    nums, dens = [], []
    for h in range(_H):
        arow = asT[:, h:h + 1, :]
        adh = ad3[:, :, h:h + 1]
        pos = (adh + arow) >= 0
        seld = jnp.where(pos, jnp.exp(adh), jnp.exp(0.2 * adh))
        sels = jnp.where(pos, jnp.exp(arow), jnp.exp(0.2 * arow))
        num = acnt * seld * sels
        nums.append(num)
        dens.append(jnp.sum(num, axis=2, keepdims=True))
    recc = 1.0 / (jnp.concatenate(dens, axis=2) + 1e-16)
    hmask = _hmasks()
    r8 = jnp.concatenate(hmask, axis=0)
    outs = []
    for b in range(nb):
        if npad == n:
            xpb = xp3[b]
        else:
            xpb = jnp.concatenate(
                [xp3[b], jnp.zeros((npad - n, _G), jnp.float32)], axis=0)
        o = None
        for h in range(_H):
            oh = jnp.dot(nums[h][b], xpb,
                         preferred_element_type=jnp.float32) * hmask[h]
            o = oh if o is None else o + oh
        r = jnp.dot(recc[b], r8, preferred_element_type=jnp.float32)
        outs.append(_leaky(o * r + bias))
    return outs


def _spatial_kernel(x_ref, acnt_ref, wf_ref, bf_ref, mss_ref, mds_ref,
                    bias_ref, out_ref):
    xb = x_ref[0]
    xp = jnp.dot(xb.reshape(_TT * _CP, _F), wf_ref[...],
                 preferred_element_type=jnp.float32) + bf_ref[...]
    asm = jnp.dot(xp, mss_ref[...], preferred_element_type=jnp.float32)
    ad = jnp.dot(xp, mds_ref[...], preferred_element_type=jnp.float32)
    xp3 = xp.reshape(_TT, _CP, _G)
    outs = _gat_tile(xp3, asm.reshape(_TT, _CP, _H), ad.reshape(_TT, _CP, _H),
                     acnt_ref[...], bias_ref[...], _TT, _CP, _T)
    out_ref[0] = jnp.concatenate([g[:, None, :] for g in outs], axis=1)


def _temporal_kernel(gs_ref, acnt_ref, wt_ref, mst_ref, mdt_ref, bias_ref,
                     out_ref):
    xin = gs_ref[0]
    xp = jnp.dot(xin.reshape(_CCT * _T, _G), wt_ref[...],
                 preferred_element_type=jnp.float32)
    asm = jnp.dot(xp, mst_ref[...], preferred_element_type=jnp.float32)
    ad = jnp.dot(xp, mdt_ref[...], preferred_element_type=jnp.float32)
    xp3 = xp.reshape(_CCT, _T, _G)
    outs = _gat_tile(xp3, asm.reshape(_CCT, _T, _H), ad.reshape(_CCT, _T, _H),
                     acnt_ref[...], bias_ref[...], _CCT, _T, _T)
    out_ref[0] = jnp.concatenate(outs, axis=1)


def _shift_rows(x, s):
    if s == 0:
        return x
    return jnp.concatenate(
        [jnp.zeros((s, x.shape[1]), x.dtype), x[:-s]], axis=0)


def _causal_conv(xin, w_ref, b, d):
    acc = jnp.dot(_shift_rows(xin, 2 * d), w_ref[0],
                  preferred_element_type=jnp.float32)
    acc = acc + jnp.dot(_shift_rows(xin, d), w_ref[1],
                        preferred_element_type=jnp.float32)
    acc = acc + jnp.dot(xin, w_ref[2], preferred_element_type=jnp.float32)
    return acc + b


def _tcn_kernel(x_ref, w1a_ref, w1b_ref, dw_ref, w2a_ref, w2b_ref,
                b1a_ref, b1b_ref, db_ref, g1_ref, be1_ref, m1_ref, v1_ref,
                b2a_ref, b2b_ref, g2_ref, be2_ref, m2_ref, v2_ref, out_ref):
    x = x_ref[0]
    res = jnp.dot(x, dw_ref[...], preferred_element_type=jnp.float32) \
        + db_ref[...]
    h = jax.nn.relu(_causal_conv(x, w1a_ref, b1a_ref[...], 1))
    h = jax.nn.relu(_causal_conv(h, w1b_ref, b1b_ref[...], 1))
    h = h + res
    scale1 = g1_ref[...] * jax.lax.rsqrt(v1_ref[...] + 1e-5)
    h = (h - m1_ref[...]) * scale1 + be1_ref[...]
    res2 = h
    h = jax.nn.relu(_causal_conv(h, w2a_ref, b2a_ref[...], 2))
    h = jax.nn.relu(_causal_conv(h, w2b_ref, b2b_ref[...], 2))
    h = h + res2
    scale2 = g2_ref[...] * jax.lax.rsqrt(v2_ref[...] + 1e-5)
    h = (h - m2_ref[...]) * scale2 + be2_ref[...]
    out_ref[0] = jnp.max(h, axis=0, keepdims=True)


def kernel(x, spatial_edge_index, temporal_edge_index, W_spec, b_spec, Ws,
           asrc_s, adst_s, bias_s, Wt, asrc_t, adst_t, bias_t,
           tb1_w1, tb1_b1, tb1_w2, tb1_b2, tb1_dw, tb1_db,
           tb1_gamma, tb1_beta, tb1_mean, tb1_var,
           tb2_w1, tb2_b1, tb2_w2, tb2_b2,
           tb2_gamma, tb2_beta, tb2_mean, tb2_var):
    f32 = jnp.float32
    idt = spatial_edge_index.dtype

    sl_c = jnp.arange(_C, dtype=idt)
    sl_t = jnp.arange(_T, dtype=idt)
    pad_s = jnp.full((_ESP - _ES - _C,), -1, idt)
    src_s = jnp.concatenate([spatial_edge_index[0], sl_c, pad_s])
    dst_s = jnp.concatenate([spatial_edge_index[1], sl_c, pad_s])
    pad_t = jnp.full((_ETP - _ET - _T,), -1, idt)
    src_t = jnp.concatenate([temporal_edge_index[0], sl_t, pad_t])
    dst_t = jnp.concatenate([temporal_edge_index[1], sl_t, pad_t])

    a_s, a_t, wf, bf, mss, mst, mds, mdt = pl.pallas_call(
        _prep_kernel,
        out_shape=(
            jax.ShapeDtypeStruct((_CP, _T), f32),
            jax.ShapeDtypeStruct((_T, _T), f32),
            jax.ShapeDtypeStruct((_F, _G), f32),
            jax.ShapeDtypeStruct((1, _G), f32),
            jax.ShapeDtypeStruct((_G, _H), f32),
            jax.ShapeDtypeStruct((_G, _H), f32),
            jax.ShapeDtypeStruct((_G, _H), f32),
            jax.ShapeDtypeStruct((_G, _H), f32),
        ),
    )(src_s.reshape(_ESP, 1), dst_s.reshape(1, _ESP),
      src_t.reshape(_ETP, 1), dst_t.reshape(1, _ETP),
      W_spec, Ws, b_spec.reshape(1, _DS),
      asrc_s.reshape(_G, 1), asrc_t.reshape(_G, 1),
      adst_s.reshape(_G, 1), adst_t.reshape(_G, 1))

    xT = jnp.pad(jnp.transpose(x, (0, 3, 2, 1)),
                 ((0, 0), (0, 0), (0, _CP - _C), (0, 0)))
    gs = pl.pallas_call(
        _spatial_kernel,
        grid=(_B, _T // _TT),
        in_specs=[
            pl.BlockSpec((1, _TT, _CP, _F), lambda b, t: (b, t, 0, 0)),
            pl.BlockSpec((_CP, _T), lambda b, t: (0, 0)),
            pl.BlockSpec((_F, _G), lambda b, t: (0, 0)),
            pl.BlockSpec((1, _G), lambda b, t: (0, 0)),
            pl.BlockSpec((_G, _H), lambda b, t: (0, 0)),
            pl.BlockSpec((_G, _H), lambda b, t: (0, 0)),
            pl.BlockSpec((1, _G), lambda b, t: (0, 0)),
        ],
        out_specs=pl.BlockSpec((1, _CP, _TT, _G), lambda b, t: (b, 0, t, 0)),
        out_shape=jax.ShapeDtypeStruct((_B, _CP, _T, _G), f32),
    )(xT, a_s, wf, bf, mss, mds, bias_s.reshape(1, _G))

    tcnin = pl.pallas_call(
        _temporal_kernel,
        grid=(_B, _CP // _CCT),
        in_specs=[
            pl.BlockSpec((1, _CCT, _T, _G), lambda b, c: (b, c, 0, 0)),
            pl.BlockSpec((_T, _T), lambda b, c: (0, 0)),
            pl.BlockSpec((_G, _G), lambda b, c: (0, 0)),
            pl.BlockSpec((_G, _H), lambda b, c: (0, 0)),
            pl.BlockSpec((_G, _H), lambda b, c: (0, 0)),
            pl.BlockSpec((1, _G), lambda b, c: (0, 0)),
        ],
        out_specs=pl.BlockSpec((1, _T, _CCT * _G), lambda b, c: (b, 0, c)),
        out_shape=jax.ShapeDtypeStruct((_B, _T, _CHP), f32),
    )(gs, a_t, Wt, mst, mdt, bias_t.reshape(1, _G))

    zpad = jnp.zeros((3, _CHP - _CH, _O), f32)
    w1a = jnp.concatenate([jnp.transpose(tb1_w1, (2, 1, 0)), zpad], axis=1)
    w1b = jnp.transpose(tb1_w2, (2, 1, 0))
    dw = jnp.concatenate(
        [jnp.transpose(tb1_dw[:, :, 0], (1, 0)),
         jnp.zeros((_CHP - _CH, _O), f32)], axis=0)
    w2a = jnp.transpose(tb2_w1, (2, 1, 0))
    w2b = jnp.transpose(tb2_w2, (2, 1, 0))

    def row(v):
        return v.reshape(1, _O)

    def full(*shape):
        return [pl.BlockSpec(shape, lambda b: tuple(0 for _ in shape))]

    out = pl.pallas_call(
        _tcn_kernel,
        grid=(_B,),
        in_specs=(
            [pl.BlockSpec((1, _T, _CHP), lambda b: (b, 0, 0))]
            + full(3, _CHP, _O) + full(3, _O, _O) + full(_CHP, _O)
            + full(3, _O, _O) + full(3, _O, _O)
            + full(1, _O) * 13
        ),
        out_specs=pl.BlockSpec((1, 1, _O), lambda b: (b, 0, 0)),
        out_shape=jax.ShapeDtypeStruct((_B, 1, _O), f32),
    )(tcnin, w1a, w1b, dw, w2a, w2b,
      row(tb1_b1), row(tb1_b2), row(tb1_db),
      row(tb1_gamma), row(tb1_beta), row(tb1_mean), row(tb1_var),
      row(tb2_b1), row(tb2_b2),
      row(tb2_gamma), row(tb2_beta), row(tb2_mean), row(tb2_var))

    return out.reshape(_B, _O)
